# Initial kernel scaffold; baseline (speedup 1.0000x reference)
import jax
import jax.numpy as jnp
from jax.experimental import pallas as pl


def kernel(nodes, to_neighs, features_table):
    raise NotImplementedError("write your pallas kernel here")


# speedup vs baseline: 1.4227x; 1.4227x over previous
import functools

import jax
import jax.numpy as jnp
from jax import lax
from jax.experimental import pallas as pl
from jax.experimental.pallas import tpu as pltpu
from jax.experimental.pallas import tpu_sc as plsc

NC = 2
NS = 16
NW = NC * NS
L = 16
S = 32
D = 128
C = 4
BP = 10240
BPW = BP // NW
NCHUNKS = BPW // C
NVREG = D // L


@functools.partial(
    pl.kernel,
    out_type=jax.ShapeDtypeStruct((BP, D), jnp.float32),
    mesh=plsc.VectorSubcoreMesh(
        core_axis_name="c", subcore_axis_name="s",
        num_cores=NC, num_subcores=NS),
    scratch_types=[
        pltpu.VMEM((NCHUNKS, C * S), jnp.int32),
        pltpu.VMEM((C * S, D), jnp.float32),
        pltpu.VMEM((C * S, D), jnp.float32),
        pltpu.VMEM((BPW, D), jnp.float32),
        pltpu.SemaphoreType.DMA,
        pltpu.SemaphoreType.DMA,
    ],
)
def _mean_agg(idx_hbm, table_hbm, out_hbm, idx_v, buf0, buf1, out_v,
              sem0, sem1):
    wid = lax.axis_index("s") * NC + lax.axis_index("c")
    pltpu.sync_copy(idx_hbm.at[pl.ds(wid * NCHUNKS, NCHUNKS)], idx_v)

    bufs = (buf0, buf1)
    sems = (sem0, sem1)

    pltpu.async_copy(table_hbm.at[idx_v.at[0]], buf0, sem0)
    pltpu.async_copy(table_hbm.at[idx_v.at[1]], buf1, sem1)

    def compute(c, buf):
        for r in range(C):
            def body(s_, carry):
                row = r * S + s_
                return tuple(
                    a + buf[row, pl.ds(v * L, L)]
                    for v, a in enumerate(carry))
            acc = lax.fori_loop(
                0, S, body,
                tuple(jnp.zeros((L,), jnp.float32) for _ in range(NVREG)))
            orow = c * C + r
            for v in range(NVREG):
                out_v[orow, pl.ds(v * L, L)] = acc[v] * (1.0 / S)

    def outer(g, _):
        for b in range(2):
            c = g * 2 + b
            pltpu.make_async_copy(
                table_hbm.at[idx_v.at[0]], bufs[b], sems[b]).wait()
            compute(c, bufs[b])

            @pl.when(c + 2 < NCHUNKS)
            def _():
                pltpu.async_copy(
                    table_hbm.at[idx_v.at[c + 2]], bufs[b], sems[b])
        return _

    lax.fori_loop(0, NCHUNKS // 2, outer, None)
    pltpu.sync_copy(out_v, out_hbm.at[pl.ds(wid * BPW, BPW)])


def kernel(nodes, to_neighs, features_table):
    del nodes
    b = to_neighs.shape[0]
    idx = jnp.pad(to_neighs.astype(jnp.int32), ((0, BP - b), (0, 0)))
    idx2d = idx.reshape(BP * S // (C * S), C * S)
    out = _mean_agg(idx2d, features_table)
    return out[:b]


# speedup vs baseline: 1.5009x; 1.0549x over previous
import functools

import jax
import jax.numpy as jnp
from jax import lax
from jax.experimental import pallas as pl
from jax.experimental.pallas import tpu as pltpu
from jax.experimental.pallas import tpu_sc as plsc

NC = 2
NS = 16
NW = NC * NS
L = 16
S = 32
D = 128
C = 4
BP = 10240
NVREG = D // L
BPW0 = 512
BPW1 = 128
NCH0 = BPW0 // C
NCH1 = BPW1 // C


@functools.partial(
    pl.kernel,
    out_type=jax.ShapeDtypeStruct((BP, D), jnp.float32),
    mesh=plsc.VectorSubcoreMesh(
        core_axis_name="c", subcore_axis_name="s",
        num_cores=NC, num_subcores=NS),
    scratch_types=[
        pltpu.VMEM((NCH0, C * S), jnp.int32),
        pltpu.VMEM((C * S, D), jnp.float32),
        pltpu.VMEM((C * S, D), jnp.float32),
        pltpu.VMEM((BPW0, D), jnp.float32),
        pltpu.SemaphoreType.DMA,
        pltpu.SemaphoreType.DMA,
    ],
)
def _mean_agg(idx_hbm, table_hbm, out_hbm, idx_v, buf0, buf1, out_v,
              sem0, sem1):
    cid = lax.axis_index("c")
    sid = lax.axis_index("s")

    bufs = (buf0, buf1)
    sems = (sem0, sem1)

    def compute(c, buf):
        for r in range(C):
            def body(s_, carry):
                row = r * S + s_
                return tuple(
                    a + buf[row, pl.ds(v * L, L)]
                    for v, a in enumerate(carry))
            acc = lax.fori_loop(
                0, S, body,
                tuple(jnp.zeros((L,), jnp.float32) for _ in range(NVREG)))
            orow = c * C + r
            for v in range(NVREG):
                out_v[orow, pl.ds(v * L, L)] = acc[v] * (1.0 / S)

    def run(nch, out_base, idx_base):
        pltpu.sync_copy(idx_hbm.at[pl.ds(idx_base, nch)],
                        idx_v.at[pl.ds(0, nch)])
        pltpu.async_copy(table_hbm.at[idx_v.at[0]], buf0, sem0)
        pltpu.async_copy(table_hbm.at[idx_v.at[1]], buf1, sem1)

        def outer(g, carry):
            for b in range(2):
                c = g * 2 + b
                pltpu.make_async_copy(
                    table_hbm.at[idx_v.at[0]], bufs[b], sems[b]).wait()
                compute(c, bufs[b])

                @pl.when(c + 2 < nch)
                def _():
                    pltpu.async_copy(
                        table_hbm.at[idx_v.at[c + 2]], bufs[b], sems[b])
            return carry

        lax.fori_loop(0, nch // 2, outer, 0)
        pltpu.sync_copy(out_v.at[pl.ds(0, nch * C)],
                        out_hbm.at[pl.ds(out_base, nch * C)])

    @pl.when(cid == 0)
    def _():
        run(NCH0, sid * BPW0, sid * NCH0)

    @pl.when(cid == 1)
    def _():
        run(NCH1, NS * BPW0 + sid * BPW1, NS * NCH0 + sid * NCH1)


def kernel(nodes, to_neighs, features_table):
    del nodes
    b = to_neighs.shape[0]
    idx = jnp.pad(to_neighs.astype(jnp.int32), ((0, BP - b), (0, 0)))
    idx2d = idx.reshape(BP * S // (C * S), C * S)
    out = _mean_agg(idx2d, features_table)
    return out[:b]
